# Initial kernel scaffold; baseline (speedup 1.0000x reference)
import jax
import jax.numpy as jnp
from jax.experimental import pallas as pl


def kernel(braids, initial_state, thetas, ln_gamma, ln_beta, w1, b1, w2, b2, w3, b3):
    raise NotImplementedError("write your pallas kernel here")


# speedup vs baseline: 10.1258x; 10.1258x over previous
import jax
import jax.numpy as jnp
from jax.experimental import pallas as pl
from jax.experimental.pallas import tpu as pltpu

NUM_STRANDS = 4
HIDDEN = 128
LAYERS = 2
B = 1024
L = 20
BS = B // 128


def _compose_kernel(braids_ref, thetas_ref, out_ref):
    for layer in range(LAYERS):
        cos_t = [jnp.cos(thetas_ref[layer, p]) for p in range(3)]
        sin_t = [jnp.sin(thetas_ref[layer, p]) for p in range(3)]
        one = jnp.ones((BS, 128), jnp.float32)
        zero = jnp.zeros((BS, 128), jnp.float32)
        m = [one if (e % 5 == 0) else zero for e in range(16)]
        for t in range(L):
            gen = braids_ref[t]
            mask = gen != 0
            p = jnp.abs(gen) - 1
            sgn = jnp.sign(gen).astype(jnp.float32)
            cth = jnp.where(p == 0, cos_t[0],
                            jnp.where(p == 1, cos_t[1], cos_t[2]))
            sth = sgn * jnp.where(p == 0, sin_t[0],
                                  jnp.where(p == 1, sin_t[1], sin_t[2]))
            for pp in range(3):
                sel = mask & (p == pp)
                c = jnp.where(sel, cth, 1.0)
                s = jnp.where(sel, sth, 0.0)
                for j in range(4):
                    u = m[pp * 4 + j]
                    v = m[(pp + 1) * 4 + j]
                    m[pp * 4 + j] = u * c - v * s
                    m[(pp + 1) * 4 + j] = u * s + v * c
        for e in range(16):
            out_ref[layer * 16 + e] = m[e]


def _layernorm(x, gamma, beta):
    mean = jnp.mean(x, axis=1, keepdims=True)
    cen = x - mean
    var = jnp.mean(cen * cen, axis=1, keepdims=True)
    return cen * jax.lax.rsqrt(var + 1e-5) * gamma + beta


def _dense_kernel(m0_ref, m1_ref, init_ref, g_ref, b_ref,
                  w1_ref, b1_ref, w2_ref, b2_ref, w3_ref, b3_ref, out_ref):
    s1 = []
    for i in range(4):
        acc = m0_ref[:, 4 * i:4 * i + 1] * init_ref[0:1, :]
        for j in range(1, 4):
            acc = acc + m0_ref[:, 4 * i + j:4 * i + j + 1] * init_ref[j:j + 1, :]
        s1.append(acc)
    g0 = g_ref[0:1, :]
    b0 = b_ref[0:1, :]
    s1 = [_layernorm(x, g0, b0) for x in s1]
    s2 = []
    for i in range(4):
        acc = m1_ref[:, 4 * i:4 * i + 1] * s1[0]
        for j in range(1, 4):
            acc = acc + m1_ref[:, 4 * i + j:4 * i + j + 1] * s1[j]
        s2.append(acc)
    g1 = g_ref[1:2, :]
    b1n = b_ref[1:2, :]
    s2 = [_layernorm(x, g1, b1n) for x in s2]
    h1 = b1_ref[0:1, :]
    for j in range(4):
        h1 = h1 + jnp.dot(s2[j], w1_ref[128 * j:128 * (j + 1), :],
                          preferred_element_type=jnp.float32)
    h1 = jnp.maximum(h1, 0.0)
    h2 = jnp.dot(h1, w2_ref[...], preferred_element_type=jnp.float32)
    h2 = jnp.maximum(h2 + b2_ref[0:1, :], 0.0)
    z0 = jnp.sum(h2 * w3_ref[0:1, :], axis=1, keepdims=True) + b3_ref[0]
    z1 = jnp.sum(h2 * w3_ref[1:2, :], axis=1, keepdims=True) + b3_ref[1]
    out_ref[:, 0:1] = jax.nn.sigmoid(z0)
    out_ref[:, 1:2] = z1


def kernel(braids, initial_state, thetas, ln_gamma, ln_beta,
           w1, b1, w2, b2, w3, b3):
    braids_t = braids.T.reshape(L, BS, 128)
    ms = pl.pallas_call(
        _compose_kernel,
        out_shape=jax.ShapeDtypeStruct((2 * 16, BS, 128), jnp.float32),
        in_specs=[
            pl.BlockSpec(memory_space=pltpu.VMEM),
            pl.BlockSpec(memory_space=pltpu.SMEM),
        ],
        out_specs=pl.BlockSpec(memory_space=pltpu.VMEM),
    )(braids_t, thetas)
    ms = jnp.transpose(ms.reshape(2, 16, BS, 128), (0, 2, 3, 1)).reshape(2, B, 16)
    out = pl.pallas_call(
        _dense_kernel,
        out_shape=jax.ShapeDtypeStruct((B, 2), jnp.float32),
        in_specs=[
            pl.BlockSpec(memory_space=pltpu.VMEM),
            pl.BlockSpec(memory_space=pltpu.VMEM),
            pl.BlockSpec(memory_space=pltpu.VMEM),
            pl.BlockSpec(memory_space=pltpu.VMEM),
            pl.BlockSpec(memory_space=pltpu.VMEM),
            pl.BlockSpec(memory_space=pltpu.VMEM),
            pl.BlockSpec(memory_space=pltpu.VMEM),
            pl.BlockSpec(memory_space=pltpu.VMEM),
            pl.BlockSpec(memory_space=pltpu.VMEM),
            pl.BlockSpec(memory_space=pltpu.VMEM),
            pl.BlockSpec(memory_space=pltpu.SMEM),
        ],
        out_specs=pl.BlockSpec(memory_space=pltpu.VMEM),
    )(ms[0], ms[1], initial_state, ln_gamma, ln_beta,
      w1.T, b1.reshape(1, 128), w2.T, b2.reshape(1, 64), w3, b3)
    return out[:, 0], out[:, 1]
